# Initial kernel scaffold; baseline (speedup 1.0000x reference)
import jax
import jax.numpy as jnp
from jax.experimental import pallas as pl


def kernel(x, w1s, w2s):
    raise NotImplementedError("write your pallas kernel here")


# speedup vs baseline: 3.9320x; 3.9320x over previous
import functools

import jax
import jax.numpy as jnp
from jax import lax
from jax.experimental import pallas as pl
from jax.experimental.pallas import tpu as pltpu


def _fff_body(depth, bt, n_pad, x_ref, w1_ref, w2_ref, y_ref, L_ref, C_ref):
    x_b = x_ref[:]
    L_ref[:] = lax.dot_general(
        x_b, w1_ref[:],
        dimension_numbers=(((1,), (1,)), ((), ())),
        precision=lax.Precision.HIGHEST,
        preferred_element_type=jnp.float32,
    )
    C_ref[:] = jnp.zeros((bt, n_pad), jnp.float32)
    n = jnp.zeros((bt, 1), jnp.int32)
    for d in range(depth):
        lo = 2 ** d - 1
        width = 2 ** d
        Ld = L_ref[:, lo:lo + width]
        off = n - lo
        io = lax.broadcasted_iota(jnp.int32, (bt, width), 1)
        oh = io == off
        sel = jnp.where(oh, Ld, 0.0)
        C_ref[:, lo:lo + width] = sel
        lam = jnp.sum(sel, axis=1, keepdims=True)
        n = 2 * n + 1 + (lam > 0).astype(jnp.int32)
    y_ref[:] = lax.dot_general(
        C_ref[:], w2_ref[:],
        dimension_numbers=(((1,), (0,)), ((), ())),
        precision=lax.Precision.DEFAULT,
        preferred_element_type=jnp.float32,
    )


def kernel(x, w1s, w2s):
    b, n_in = x.shape
    n_nodes, _ = w1s.shape
    n_out = w2s.shape[1]
    depth = (n_nodes + 1).bit_length() - 1
    n_pad = n_nodes + 1
    w1p = jnp.pad(w1s, ((0, n_pad - n_nodes), (0, 0)))
    w2p = jnp.pad(w2s, ((0, n_pad - n_nodes), (0, 0)))
    bt = 256
    grid = b // bt
    body = functools.partial(_fff_body, depth, bt, n_pad)
    return pl.pallas_call(
        body,
        grid=(grid,),
        in_specs=[
            pl.BlockSpec((bt, n_in), lambda i: (i, 0)),
            pl.BlockSpec((n_pad, n_in), lambda i: (0, 0)),
            pl.BlockSpec((n_pad, n_out), lambda i: (0, 0)),
        ],
        out_specs=pl.BlockSpec((bt, n_out), lambda i: (i, 0)),
        out_shape=jax.ShapeDtypeStruct((b, n_out), jnp.float32),
        scratch_shapes=[
            pltpu.VMEM((bt, n_pad), jnp.float32),
            pltpu.VMEM((bt, n_pad), jnp.float32),
        ],
    )(x, w1p, w2p)


# speedup vs baseline: 5.2824x; 1.3434x over previous
import functools

import jax
import jax.numpy as jnp
from jax import lax
from jax.experimental import pallas as pl
from jax.experimental.pallas import tpu as pltpu


def _fff_body(depth, bt, n_pad, x_ref, w1_ref, w2_ref, y_ref, L_ref, C_ref):
    x_b = x_ref[:]
    half = n_pad // 2
    L_ref[:, :half] = lax.dot_general(
        x_b, w1_ref[:half, :],
        dimension_numbers=(((1,), (1,)), ((), ())),
        precision=lax.Precision.HIGHEST,
        preferred_element_type=jnp.float32,
    )
    L_ref[:, half:] = lax.dot_general(
        x_b, w1_ref[half:, :],
        dimension_numbers=(((1,), (1,)), ((), ())),
        precision=lax.Precision.DEFAULT,
        preferred_element_type=jnp.float32,
    )
    C_ref[:] = jnp.zeros((bt, n_pad), jnp.float32)
    n = jnp.zeros((bt, 1), jnp.int32)
    for d in range(depth):
        lo = 2 ** d - 1
        width = 2 ** d
        Ld = L_ref[:, lo:lo + width]
        off = n - lo
        io = lax.broadcasted_iota(jnp.int32, (bt, width), 1)
        oh = io == off
        sel = jnp.where(oh, Ld, 0.0)
        C_ref[:, lo:lo + width] = sel
        lam = jnp.sum(sel, axis=1, keepdims=True)
        n = 2 * n + 1 + (lam > 0).astype(jnp.int32)
    y_ref[:] = lax.dot_general(
        C_ref[:], w2_ref[:],
        dimension_numbers=(((1,), (0,)), ((), ())),
        precision=lax.Precision.DEFAULT,
        preferred_element_type=jnp.float32,
    )


def kernel(x, w1s, w2s):
    b, n_in = x.shape
    n_nodes, _ = w1s.shape
    n_out = w2s.shape[1]
    depth = (n_nodes + 1).bit_length() - 1
    n_pad = n_nodes + 1
    w1p = jnp.pad(w1s, ((0, n_pad - n_nodes), (0, 0)))
    w2p = jnp.pad(w2s, ((0, n_pad - n_nodes), (0, 0)))
    bt = 256
    grid = b // bt
    body = functools.partial(_fff_body, depth, bt, n_pad)
    return pl.pallas_call(
        body,
        grid=(grid,),
        in_specs=[
            pl.BlockSpec((bt, n_in), lambda i: (i, 0)),
            pl.BlockSpec((n_pad, n_in), lambda i: (0, 0)),
            pl.BlockSpec((n_pad, n_out), lambda i: (0, 0)),
        ],
        out_specs=pl.BlockSpec((bt, n_out), lambda i: (i, 0)),
        out_shape=jax.ShapeDtypeStruct((b, n_out), jnp.float32),
        scratch_shapes=[
            pltpu.VMEM((bt, n_pad), jnp.float32),
            pltpu.VMEM((bt, n_pad), jnp.float32),
        ],
    )(x, w1p, w2p)
